# baseline (speedup 1.0000x reference)
import functools

import jax
import jax.numpy as jnp
from jax import lax
from jax.experimental import pallas as pl
from jax.experimental.pallas import tpu as pltpu
from jax.experimental.pallas import tpu_sc as plsc

HIDDEN = 768
EPS = 1e-5
NUM_WORKERS = 32
NCHUNKS = 4


def _sc_gather(table, idx):
    _, D = table.shape
    B = idx.shape[0]
    b_per_w = B // NUM_WORKERS
    C = 128
    n_chunks = b_per_w // C
    mesh = plsc.VectorSubcoreMesh(core_axis_name="c", subcore_axis_name="s")

    @functools.partial(
        pl.kernel, mesh=mesh,
        out_type=jax.ShapeDtypeStruct((B, D), jnp.float32),
        scratch_types=[
            pltpu.VMEM((C,), jnp.int32),
            pltpu.VMEM((C, D), jnp.float32),
            pltpu.SemaphoreType.DMA,
        ],
    )
    def k(table_hbm, idx_hbm, out_hbm, idx_v, rows_v, sem):
        wid = lax.axis_index("s") * 2 + lax.axis_index("c")
        base = wid * b_per_w

        def body(i, carry):
            off = base + i * C
            pltpu.sync_copy(idx_hbm.at[pl.ds(off, C)], idx_v)
            pltpu.async_copy(table_hbm.at[idx_v], rows_v, sem).wait()
            pltpu.sync_copy(rows_v, out_hbm.at[pl.ds(off, C)])
            return carry

        lax.fori_loop(0, n_chunks, body, 0)

    return k(table, idx)


def _tc_layernorm(x, pos_emb, tt3, type_emb, gamma2, beta2):
    BATCH, SEQ, _ = x.shape
    BB = 4

    def body(x_ref, pos_ref, tt_ref, type_ref, g_ref, b_ref, o_ref):
        pos = pos_ref[...]
        t0 = type_ref[0]
        t1 = type_ref[1]
        g = g_ref[0]
        bb = b_ref[0]
        for i in range(BB):
            xb = x_ref[i]
            ttc = tt_ref[i]
            e = xb + pos + (t0[None, :] * (1.0 - ttc) + t1[None, :] * ttc)
            mean = jnp.mean(e, axis=-1, keepdims=True)
            c = e - mean
            var = jnp.mean(c * c, axis=-1, keepdims=True)
            o_ref[i] = c * lax.rsqrt(var + EPS) * g[None, :] + bb[None, :]

    return pl.pallas_call(
        body,
        grid=(BATCH // BB,),
        in_specs=[
            pl.BlockSpec((BB, SEQ, HIDDEN), lambda b: (b, 0, 0)),
            pl.BlockSpec((SEQ, HIDDEN), lambda b: (0, 0)),
            pl.BlockSpec((BB, SEQ, 1), lambda b: (b, 0, 0)),
            pl.BlockSpec((2, HIDDEN), lambda b: (0, 0)),
            pl.BlockSpec((1, HIDDEN), lambda b: (0, 0)),
            pl.BlockSpec((1, HIDDEN), lambda b: (0, 0)),
        ],
        out_specs=pl.BlockSpec((BB, SEQ, HIDDEN), lambda b: (b, 0, 0)),
        out_shape=jax.ShapeDtypeStruct((BATCH, SEQ, HIDDEN), jnp.float32),
    )(x, pos_emb, tt3, type_emb, gamma2, beta2)


def kernel(input_ids, token_type_ids, word_emb, pos_emb, type_emb, gamma, beta):
    B, S = input_ids.shape
    ids = input_ids.astype(jnp.int32)
    tt3 = token_type_ids.reshape(B, S, 1).astype(jnp.float32)
    g2 = gamma.reshape(1, HIDDEN)
    b2 = beta.reshape(1, HIDDEN)
    bc = B // NCHUNKS

    gathered = [
        _sc_gather(word_emb, ids[c * bc:(c + 1) * bc].reshape(-1))
        for c in range(NCHUNKS)
    ]
    outs = [
        _tc_layernorm(gathered[c].reshape(bc, S, HIDDEN), pos_emb,
                      tt3[c * bc:(c + 1) * bc], type_emb, g2, b2)
        for c in range(NCHUNKS)
    ]
    return jnp.concatenate(outs, axis=0)


# speedup vs baseline: 1.3963x; 1.3963x over previous
import functools

import jax
import jax.numpy as jnp
from jax import lax
from jax.experimental import pallas as pl
from jax.experimental.pallas import tpu as pltpu
from jax.experimental.pallas import tpu_sc as plsc

HIDDEN = 768
EPS = 1e-5
NUM_WORKERS = 32


def _sc_gather(table, idx):
    _, D = table.shape
    B = idx.shape[0]
    b_per_w = B // NUM_WORKERS
    C = 64
    n_ch = b_per_w // C
    mesh = plsc.VectorSubcoreMesh(core_axis_name="c", subcore_axis_name="s")

    @functools.partial(
        pl.kernel, mesh=mesh,
        out_type=jax.ShapeDtypeStruct((B, D), jnp.float32),
        scratch_types=[
            pltpu.VMEM((b_per_w,), jnp.int32),
            pltpu.VMEM((C, D), jnp.float32),
            pltpu.VMEM((C, D), jnp.float32),
            pltpu.SemaphoreType.DMA,
            pltpu.SemaphoreType.DMA,
            pltpu.SemaphoreType.DMA,
            pltpu.SemaphoreType.DMA,
        ],
    )
    def k(table_hbm, idx_hbm, out_hbm, idx_v, rows0, rows1, sg0, sg1, so0, so1):
        wid = lax.axis_index("s") * 2 + lax.axis_index("c")
        base = wid * b_per_w
        pltpu.sync_copy(idx_hbm.at[pl.ds(base, b_per_w)], idx_v)
        bufs = ((rows0, sg0, so0), (rows1, sg1, so1))

        def idx_sl(ch):
            return idx_v.at[pl.ds(ch * C, C)]

        pltpu.async_copy(table_hbm.at[idx_sl(0)], rows0, sg0)

        def pair(g, _):
            for j in range(2):
                ch = 2 * g + j
                rows, sg, so = bufs[j]
                rows_n, sg_n, so_n = bufs[1 - j]
                pltpu.make_async_copy(table_hbm.at[idx_sl(ch)], rows, sg).wait()

                @pl.when(ch >= 1)
                def _():
                    pltpu.make_async_copy(
                        rows_n, out_hbm.at[pl.ds(base + (ch - 1) * C, C)],
                        so_n).wait()

                chn = jnp.minimum(ch + 1, n_ch - 1)
                pltpu.async_copy(table_hbm.at[idx_sl(chn)], rows_n, sg_n)
                pltpu.async_copy(
                    rows, out_hbm.at[pl.ds(base + ch * C, C)], so)
            return 0

        lax.fori_loop(0, n_ch // 2, pair, 0)

        pltpu.make_async_copy(table_hbm.at[idx_sl(n_ch - 1)], rows0, sg0).wait()
        pltpu.make_async_copy(
            rows1, out_hbm.at[pl.ds(base + (n_ch - 1) * C, C)], so1).wait()

    return k(table, idx)


def _tc_layernorm(x, pos_emb, tt3, type_emb, gamma2, beta2):
    BATCH, SEQ, _ = x.shape
    BB = 4

    def body(x_ref, pos_ref, tt_ref, type_ref, g_ref, b_ref, o_ref):
        pos = pos_ref[...]
        t0 = type_ref[0]
        t1 = type_ref[1]
        g = g_ref[0]
        bb = b_ref[0]
        for i in range(BB):
            xb = x_ref[i]
            ttc = tt_ref[i]
            e = xb + pos + (t0[None, :] * (1.0 - ttc) + t1[None, :] * ttc)
            mean = jnp.mean(e, axis=-1, keepdims=True)
            c = e - mean
            var = jnp.mean(c * c, axis=-1, keepdims=True)
            o_ref[i] = c * lax.rsqrt(var + EPS) * g[None, :] + bb[None, :]

    return pl.pallas_call(
        body,
        grid=(BATCH // BB,),
        in_specs=[
            pl.BlockSpec((BB, SEQ, HIDDEN), lambda b: (b, 0, 0)),
            pl.BlockSpec((SEQ, HIDDEN), lambda b: (0, 0)),
            pl.BlockSpec((BB, SEQ, 1), lambda b: (b, 0, 0)),
            pl.BlockSpec((2, HIDDEN), lambda b: (0, 0)),
            pl.BlockSpec((1, HIDDEN), lambda b: (0, 0)),
            pl.BlockSpec((1, HIDDEN), lambda b: (0, 0)),
        ],
        out_specs=pl.BlockSpec((BB, SEQ, HIDDEN), lambda b: (b, 0, 0)),
        out_shape=jax.ShapeDtypeStruct((BATCH, SEQ, HIDDEN), jnp.float32),
    )(x, pos_emb, tt3, type_emb, gamma2, beta2)


def kernel(input_ids, token_type_ids, word_emb, pos_emb, type_emb, gamma, beta):
    B, S = input_ids.shape
    ids = input_ids.reshape(-1).astype(jnp.int32)
    gathered = _sc_gather(word_emb, ids)
    x = gathered.reshape(B, S, HIDDEN)
    tt3 = token_type_ids.reshape(B, S, 1).astype(jnp.float32)
    return _tc_layernorm(
        x, pos_emb, tt3, type_emb,
        gamma.reshape(1, HIDDEN), beta.reshape(1, HIDDEN),
    )
